# baseline (speedup 1.0000x reference)
import functools

import jax
import jax.numpy as jnp
from jax import lax
from jax.experimental import pallas as pl
from jax.experimental.pallas import tpu as pltpu
from jax.experimental.pallas import tpu_sc as plsc


def _bitonic(vs):
    if len(vs) == 1:
        v = vs[0]
        return [None if v is None else lax.sort(v)]
    h = len(vs) // 2
    lo, hi = [], []
    for a, b in zip(vs[:h], vs[h:]):
        if a is None and b is None:
            lo.append(None)
            hi.append(None)
        elif a is None:
            lo.append(b)
            hi.append(None)
        elif b is None:
            lo.append(a)
            hi.append(None)
        else:
            lo.append(jnp.minimum(a, b))
            hi.append(jnp.maximum(a, b))
    return _bitonic(lo) + _bitonic(hi)


def _merge(x, y):
    rev_y = [None if v is None else lax.rev(v, (0,)) for v in reversed(y)]
    return _bitonic(x + rev_y)


@functools.lru_cache(maxsize=None)
def _build_sc_col_sort(F, B):
    info = plsc.get_sparse_core_info()
    NC, NS, L = info.num_cores, info.num_subcores, info.num_lanes
    NW = NC * NS
    assert B % NW == 0
    cols_w = B // NW
    nfull = F // L
    rem = F - nfull * L
    mesh = plsc.VectorSubcoreMesh(core_axis_name="c", subcore_axis_name="s")

    @functools.partial(
        pl.kernel,
        mesh=mesh,
        compiler_params=pltpu.CompilerParams(needs_layout_passes=False),
        out_type=jax.ShapeDtypeStruct((F, B), jnp.float32),
        scratch_types=[pltpu.VMEM((F, cols_w), jnp.float32)],
    )
    def k(x_hbm, out_hbm, xio):
        wid = lax.axis_index("s") * NC + lax.axis_index("c")
        b0 = wid * cols_w
        pltpu.sync_copy(x_hbm.at[:, pl.ds(b0, cols_w)], xio)
        lane = lax.iota(jnp.int32, L)
        fvecs = [lane + j * L for j in range(nfull)]
        ftail = jnp.minimum(lane + nfull * L, F - 1)
        tmask = lane < rem

        @plsc.parallel_loop(0, cols_w, unroll=4)
        def body(b):
            bvec = jnp.broadcast_to(b, (L,))
            regs = [plsc.load_gather(xio, [fv, bvec]) for fv in fvecs]
            tail = plsc.load_gather(xio, [ftail, bvec])
            regs.append(jnp.where(tmask, tail, jnp.inf))
            s = [lax.sort(v) for v in regs]
            a = _merge([s[0]], [s[1]])
            c = _merge([s[2]], [s[3]])
            d = _merge([s[4]], [s[5]])
            e = _merge(a, c)
            f = _merge(d, [s[6], None])
            g = _merge(e, f)
            for j in range(nfull):
                plsc.store_scatter(xio, [fvecs[j], bvec], g[j])
            plsc.store_scatter(xio, [ftail, bvec], g[nfull], mask=tmask)

        pltpu.sync_copy(xio, out_hbm.at[:, pl.ds(b0, cols_w)])

    return k


def kernel(batchX, W):
    B, F = batchX.shape
    yT = _build_sc_col_sort(F, B)(batchX.T)
    return yT.T.reshape(B, 1, 10, 10)


# speedup vs baseline: 1.5319x; 1.5319x over previous
import functools

import jax
import jax.numpy as jnp
from jax import lax
from jax.experimental import pallas as pl
from jax.experimental.pallas import tpu as pltpu
from jax.experimental.pallas import tpu_sc as plsc


def _bitonic(vs):
    if len(vs) == 1:
        v = vs[0]
        return [None if v is None else lax.sort(v)]
    h = len(vs) // 2
    lo, hi = [], []
    for a, b in zip(vs[:h], vs[h:]):
        if a is None and b is None:
            lo.append(None)
            hi.append(None)
        elif a is None:
            lo.append(b)
            hi.append(None)
        elif b is None:
            lo.append(a)
            hi.append(None)
        else:
            lo.append(jnp.minimum(a, b))
            hi.append(jnp.maximum(a, b))
    return _bitonic(lo) + _bitonic(hi)


def _merge(x, y):
    rev_y = [None if v is None else lax.rev(v, (0,)) for v in reversed(y)]
    return _bitonic(x + rev_y)


@functools.lru_cache(maxsize=None)
def _build_sc_col_sort(F, B):
    info = plsc.get_sparse_core_info()
    NC, NS, L = info.num_cores, info.num_subcores, info.num_lanes
    NW = NC * NS
    assert B % NW == 0
    cols_w = B // NW
    nfull = F // L
    rem = F - nfull * L
    cstride = F + 5
    assert cstride % 2 == 1 and cstride >= F
    nchunks = cols_w // L
    mesh = plsc.VectorSubcoreMesh(core_axis_name="c", subcore_axis_name="s")

    @functools.partial(
        pl.kernel,
        mesh=mesh,
        compiler_params=pltpu.CompilerParams(needs_layout_passes=False),
        out_type=jax.ShapeDtypeStruct((F, B), jnp.float32),
        scratch_types=[
            pltpu.VMEM((F, cols_w), jnp.float32),
            pltpu.VMEM((cols_w * cstride + L,), jnp.float32),
        ],
    )
    def k(x_hbm, out_hbm, xio, skw):
        wid = lax.axis_index("s") * NC + lax.axis_index("c")
        b0 = wid * cols_w
        pltpu.sync_copy(x_hbm.at[:, pl.ds(b0, cols_w)], xio)
        lane = lax.iota(jnp.int32, L)
        lstride = lane * cstride
        tmask = lane >= L - rem

        @plsc.parallel_loop(0, nchunks, unroll=1)
        def t_in(ci):
            cbase = ci * (L * cstride)
            for f in range(F):
                v = xio[f, pl.ds(ci * L, L)]
                plsc.store_scatter(skw, [lstride + (cbase + f)], v)

        @plsc.parallel_loop(0, cols_w, unroll=4)
        def body(b):
            base = b * cstride
            regs = [skw[pl.ds(base + j * L, L)] for j in range(nfull)]
            tail = skw[pl.ds(base + F - L, L)]
            regs.append(jnp.where(tmask, tail, jnp.inf))
            s = [lax.sort(v) for v in regs]
            a = _merge([s[0]], [s[1]])
            c = _merge([s[2]], [s[3]])
            d = _merge([s[4]], [s[5]])
            e = _merge(a, c)
            f = _merge(d, [s[6], None])
            g = _merge(e, f)
            for j in range(nfull):
                skw[pl.ds(base + j * L, L)] = g[j]
            plsc.store_scatter(
                skw, [lane + (base + nfull * L)], g[nfull], mask=lane < rem
            )

        @plsc.parallel_loop(0, nchunks, unroll=1)
        def t_out(ci):
            cbase = ci * (L * cstride)
            for f in range(F):
                v = plsc.load_gather(skw, [lstride + (cbase + f)])
                xio[f, pl.ds(ci * L, L)] = v

        pltpu.sync_copy(xio, out_hbm.at[:, pl.ds(b0, cols_w)])

    return k


def kernel(batchX, W):
    B, F = batchX.shape
    yT = _build_sc_col_sort(F, B)(batchX.T)
    return yT.T.reshape(B, 1, 10, 10)


# speedup vs baseline: 2.1865x; 1.4273x over previous
import functools

import jax
import jax.numpy as jnp
from jax import lax
from jax.experimental import pallas as pl
from jax.experimental.pallas import tpu as pltpu
from jax.experimental.pallas import tpu_sc as plsc


def _bitonic(vs):
    if len(vs) == 1:
        v = vs[0]
        return [None if v is None else lax.sort(v)]
    h = len(vs) // 2
    lo, hi = [], []
    for a, b in zip(vs[:h], vs[h:]):
        if a is None and b is None:
            lo.append(None)
            hi.append(None)
        elif a is None:
            lo.append(b)
            hi.append(None)
        elif b is None:
            lo.append(a)
            hi.append(None)
        else:
            lo.append(jnp.minimum(a, b))
            hi.append(jnp.maximum(a, b))
    return _bitonic(lo) + _bitonic(hi)


def _merge(x, y):
    rev_y = [None if v is None else lax.rev(v, (0,)) for v in reversed(y)]
    return _bitonic(x + rev_y)


@functools.lru_cache(maxsize=None)
def _build_sc_col_sort(F, B):
    info = plsc.get_sparse_core_info()
    NC, NS, L = info.num_cores, info.num_subcores, info.num_lanes
    NW = NC * NS
    assert B % NW == 0
    cols_w = B // NW
    nfull = F // L
    rem = F - nfull * L
    cstride = F + 5
    assert cstride % 2 == 1 and cstride >= F
    nchunks = cols_w // L
    mesh = plsc.VectorSubcoreMesh(core_axis_name="c", subcore_axis_name="s")

    @functools.partial(
        pl.kernel,
        mesh=mesh,
        compiler_params=pltpu.CompilerParams(needs_layout_passes=False),
        out_type=jax.ShapeDtypeStruct((F * B,), jnp.float32),
        scratch_types=[
            pltpu.VMEM((F, cols_w), jnp.float32),
            pltpu.VMEM((cols_w * cstride + L,), jnp.float32),
            pltpu.SemaphoreType.DMA,
        ],
    )
    def k(x_hbm, out_hbm, xio, skw, sem):
        wid = lax.axis_index("s") * NC + lax.axis_index("c")
        b0 = wid * cols_w
        pltpu.sync_copy(x_hbm.at[:, pl.ds(b0, cols_w)], xio)
        lane = lax.iota(jnp.int32, L)
        lstride = lane * cstride
        tmask = lane >= L - rem

        @plsc.parallel_loop(0, nchunks, unroll=1)
        def t_in(ci):
            cbase = ci * (L * cstride)
            for f in range(F):
                v = xio[f, pl.ds(ci * L, L)]
                plsc.store_scatter(skw, [lstride + (cbase + f)], v)

        @plsc.parallel_loop(0, cols_w, unroll=4)
        def body(b):
            base = b * cstride
            regs = [skw[pl.ds(base + j * L, L)] for j in range(nfull)]
            tail = skw[pl.ds(base + F - L, L)]
            regs.append(jnp.where(tmask, tail, jnp.inf))
            s = [lax.sort(v) for v in regs]
            a = _merge([s[0]], [s[1]])
            c = _merge([s[2]], [s[3]])
            d = _merge([s[4]], [s[5]])
            e = _merge(a, c)
            f = _merge(d, [s[6], None])
            g = _merge(e, f)
            for j in range(nfull):
                skw[pl.ds(base + j * L, L)] = g[j]
            plsc.store_scatter(
                skw, [lane + (base + nfull * L)], g[nfull], mask=lane < rem
            )

        @plsc.parallel_loop(0, nchunks, unroll=1)
        def t_out(ci):
            cbase = ci * (L * cstride)
            for f in range(F):
                v = plsc.load_gather(skw, [lstride + (cbase + f)])
                xio[f, pl.ds(ci * L, L)] = v

        copies = [
            pltpu.async_copy(
                xio.at[f], out_hbm.at[pl.ds(f * B + b0, cols_w)], sem
            )
            for f in range(F)
        ]
        for cp in copies:
            cp.wait()

    return k


def kernel(batchX, W):
    B, F = batchX.shape
    yflat = _build_sc_col_sort(F, B)(batchX.T)
    return yflat.reshape(1, 10, 10, B).transpose(3, 0, 1, 2)
